# baseline (speedup 1.0000x reference)
import functools

import jax
import jax.numpy as jnp
from jax import lax
from jax.experimental import pallas as pl
from jax.experimental.pallas import tpu as pltpu
from jax.experimental.pallas import tpu_sc as plsc

B = 16384
L = 50
V = 100000
D = 200
NT = 16
ND = 7
NDOM = 10000
FPAD = 224

_INFO = plsc.get_sparse_core_info()
_NC = _INFO.num_cores
_NS = _INFO.num_subcores
NW = _NC * _NS
RPW = B // NW
GPW = RPW // 16

_COFF = tuple(c * 16 for c in range(12)) + (184,)


def _sc_body(tok_hbm, tokf_hbm, len_hbm, tid_hbm, did_hbm, dom_hbm, hrs_hbm,
             kar_hbm, des_hbm, w2vf_hbm, typet_hbm, dayt_hbm, domt_hbm,
             out_hbm, tok_v, tokf_v, emb_v, out_v, lens_v, tids_v, dids_v,
             doms_v, hrs_v, kar_v, des_v, typet_v, dayt_v, domt_v, row0_v,
             sem):
    wid = lax.axis_index("s") * _NC + lax.axis_index("c")
    base = wid * RPW
    w2v_hbm = w2vf_hbm

    pltpu.sync_copy(len_hbm.at[pl.ds(base, RPW)], lens_v)
    pltpu.sync_copy(tid_hbm.at[pl.ds(base, RPW)], tids_v)
    pltpu.sync_copy(did_hbm.at[pl.ds(base, RPW)], dids_v)
    pltpu.sync_copy(dom_hbm.at[pl.ds(base, RPW)], doms_v)
    pltpu.sync_copy(hrs_hbm.at[pl.ds(base, RPW)], hrs_v)
    pltpu.sync_copy(kar_hbm.at[pl.ds(base, RPW)], kar_v)
    pltpu.sync_copy(des_hbm.at[pl.ds(base, RPW)], des_v)
    pltpu.sync_copy(typet_hbm, typet_v)
    pltpu.sync_copy(dayt_hbm, dayt_v)
    pltpu.sync_copy(domt_hbm, domt_v)
    pltpu.sync_copy(w2v_hbm.at[pl.ds(0, 1), :], row0_v)

    iota16 = lax.iota(jnp.int32, 16)
    f32 = jnp.float32

    def zero_body(i, c):
        out_v[pl.ds(i * 16, 16)] = jnp.zeros((16,), f32)
        return c

    lax.fori_loop(0, 256, zero_body, 0)
    rowpart16 = (iota16 // 8) * 2048 + (iota16 % 8) * 128

    def group_body(g, carry):
        gbase = base + g * 16
        off = g * 16
        pltpu.sync_copy(tok_hbm.at[pl.ds(gbase, 16), :], tok_v)
        pltpu.sync_copy(tokf_hbm.at[pl.ds(gbase * L, 16 * L)], tokf_v)
        inv16 = 1.0 / jnp.maximum(
            lens_v[pl.ds(off, 16)].astype(jnp.float32), 1.0)

        pltpu.async_copy(w2v_hbm.at[tok_v.at[0]], emb_v.at[0], sem.at[0])
        pltpu.async_copy(w2v_hbm.at[tok_v.at[1]], emb_v.at[1], sem.at[1])

        def row_body(r, carry2):
            par = lax.rem(r, 3)
            nxt = lax.rem(r + 2, 3)

            @pl.when(r < 14)
            def _():
                pltpu.async_copy(w2v_hbm.at[tok_v.at[r + 2]], emb_v.at[nxt],
                                 sem.at[nxt])

            pltpu.make_async_copy(w2v_hbm.at[tok_v.at[r]], emb_v.at[par],
                                  sem.at[par]).wait()

            def acc_body(j, acc):
                return tuple(acc[c] + emb_v[par, j, pl.ds(_COFF[c], 16)]
                             for c in range(13))

            acc = lax.fori_loop(
                0, L, acc_body,
                tuple(jnp.zeros((16,), f32) for _ in range(13)))

            n0 = jnp.zeros((16,), jnp.int32)
            rsplat = jnp.full((16,), r, jnp.int32)
            for c in range(4):
                colpos = c * 16 + iota16
                colidx = jnp.minimum(colpos, L - 1)
                vals = plsc.load_gather(tokf_v, [rsplat * L + colidx])
                n0 = n0 + plsc.all_reduce_population_count(
                    (vals == 0) & (colpos < L))
            n0f = n0.astype(f32)
            inv = inv16.at[rsplat].get(mode="promise_in_bounds")

            rowoff = (r // 8) * 2048 + lax.rem(r, 8) * 128
            out_v[pl.ds(rowoff + 1024 + 72, 16)] = jnp.zeros((16,), f32)
            out_v[pl.ds(rowoff + 1024 + 88, 16)] = jnp.zeros((16,), f32)
            for c in range(13):
                res = (acc[c] - n0f * row0_v[0, pl.ds(_COFF[c], 16)]) * inv
                dbase = (1024 + _COFF[c] - 128) if _COFF[c] >= 128 \
                    else _COFF[c]
                out_v[pl.ds(rowoff + dbase, 16)] = res
            return carry2

        lax.fori_loop(0, 16, row_body, 0)

        tid = tids_v[pl.ds(off, 16)]
        did = dids_v[pl.ds(off, 16)]
        dmid = doms_v[pl.ds(off, 16)]
        for c in range(8):
            v = plsc.load_gather(typet_v, [tid * 8 + c])
            plsc.store_scatter(out_v, [rowpart16 + (896 + 200 + c)], v)
        for c in range(3):
            v = plsc.load_gather(dayt_v, [did * 3 + c])
            plsc.store_scatter(out_v, [rowpart16 + (896 + 208 + c)], v)
        for c in range(8):
            v = plsc.load_gather(domt_v, [dmid * 8 + c])
            plsc.store_scatter(out_v, [rowpart16 + (896 + 211 + c)], v)
        plsc.store_scatter(out_v, [rowpart16 + (896 + 219)],
                           hrs_v[pl.ds(off, 16)])
        plsc.store_scatter(out_v, [rowpart16 + (896 + 220)],
                           kar_v[pl.ds(off, 16)])
        plsc.store_scatter(out_v, [rowpart16 + (896 + 221)],
                           des_v[pl.ds(off, 16)])

        pltpu.sync_copy(out_v, out_hbm.at[pl.ds(gbase * 256, 4096)])
        return carry

    lax.fori_loop(0, GPW, group_body, 0)


def _sc_pool(tok, lens, tids, dids, doms, hours, karmas, desc,
             w2v, typet, dayt, domt):
    mesh = plsc.VectorSubcoreMesh(core_axis_name="c", subcore_axis_name="s")
    fn = pl.kernel(
        _sc_body,
        mesh=mesh,
        compiler_params=pltpu.CompilerParams(needs_layout_passes=False,
                                             use_tc_tiling_on_sc=False),
        out_type=jax.ShapeDtypeStruct((B * 256,), jnp.float32),
        scratch_types=[
            pltpu.VMEM((16, L), jnp.int32),
            pltpu.VMEM((16 * L,), jnp.int32),
            pltpu.VMEM((3, L, D), jnp.float32),
            pltpu.VMEM((4096,), jnp.float32),
            pltpu.VMEM((RPW,), jnp.int32),
            pltpu.VMEM((RPW,), jnp.int32),
            pltpu.VMEM((RPW,), jnp.int32),
            pltpu.VMEM((RPW,), jnp.int32),
            pltpu.VMEM((RPW,), jnp.float32),
            pltpu.VMEM((RPW,), jnp.float32),
            pltpu.VMEM((RPW,), jnp.float32),
            pltpu.VMEM((NT * 8,), jnp.float32),
            pltpu.VMEM((ND * 3,), jnp.float32),
            pltpu.VMEM((NDOM * 8,), jnp.float32),
            pltpu.VMEM((1, D), jnp.float32),
            pltpu.SemaphoreType.DMA((3,)),
        ],
    )
    return fn(tok, tok.reshape(-1), lens, tids, dids, doms, hours, karmas,
              desc, w2v, typet, dayt, domt)


def _tr_body(x_ref, o_ref):
    o_ref[...] = x_ref[...].T


def _tr(w2v_t):
    blk = 2048
    return pl.pallas_call(
        _tr_body,
        grid=((V + blk - 1) // blk,),
        in_specs=[pl.BlockSpec((D, blk), lambda i: (0, i))],
        out_specs=pl.BlockSpec((blk, D), lambda i: (i, 0)),
        out_shape=jax.ShapeDtypeStruct((V, D), jnp.float32),
    )(w2v_t)


def _mlp_body(x_ref, w1_ref, b1_ref, w2_ref, b2_ref, o_ref):
    n = x_ref.shape[0]
    h2 = jnp.dot(x_ref[...], w1_ref[...],
                 preferred_element_type=jnp.float32)
    h = h2[0:n - 8, 0:64] + h2[8:n, 64:128]
    h = jnp.maximum(h + b1_ref[...][None, :], 0.0)
    o = jnp.sum(h * w2_ref[...][None, :], axis=1) + b2_ref[...]
    o_ref[...] = jnp.concatenate([o, jnp.zeros((8,), jnp.float32)])


def _mlp(fused_tiles, w1cat, b1, w2row, b2):
    blk = 4096
    return pl.pallas_call(
        _mlp_body,
        grid=(B * 2 // blk,),
        in_specs=[
            pl.BlockSpec((blk, 128), lambda i: (i, 0)),
            pl.BlockSpec((128, 128), lambda i: (0, 0)),
            pl.BlockSpec((64,), lambda i: (0,)),
            pl.BlockSpec((64,), lambda i: (0,)),
            pl.BlockSpec((1,), lambda i: (0,)),
        ],
        out_specs=pl.BlockSpec((blk,), lambda i: (i,)),
        out_shape=jax.ShapeDtypeStruct((B * 2,), jnp.float32),
    )(fused_tiles, w1cat, b1, w2row, b2)


def kernel(title_tokens, lengths, type_ids, day_ids, domain_ids, hours,
           karmas, descendants, w2v_weight, type_table, day_table,
           domain_table, W1, b1, W2, b2):
    tok = title_tokens.astype(jnp.int32)
    w2v_weight = _tr(w2v_weight.T)
    fused = _sc_pool(
        tok, lengths.astype(jnp.int32), type_ids.astype(jnp.int32),
        day_ids.astype(jnp.int32), domain_ids.astype(jnp.int32),
        hours, karmas, descendants, w2v_weight,
        type_table.reshape(-1), day_table.reshape(-1),
        domain_table.reshape(-1)).reshape(B * 2, 128)
    w1t = W1.T
    w1cat = jnp.concatenate(
        [w1t[0:128], jnp.pad(w1t[128:], ((0, 34), (0, 0)))], axis=1)
    out_raw = _mlp(fused, w1cat, b1, W2.reshape(-1), b2)
    return out_raw.reshape(B // 8, 16)[:, 0:8].reshape(-1)


# speedup vs baseline: 1.4227x; 1.4227x over previous
import functools

import jax
import jax.numpy as jnp
from jax import lax
from jax.experimental import pallas as pl
from jax.experimental.pallas import tpu as pltpu
from jax.experimental.pallas import tpu_sc as plsc

B = 16384
L = 50
V = 100000
D = 200
NT = 16
ND = 7
NDOM = 10000
FPAD = 224

_INFO = plsc.get_sparse_core_info()
_NC = _INFO.num_cores
_NS = _INFO.num_subcores
NW = _NC * _NS
RPW = B // NW
GPW = RPW // 16


def _sc_body(tok_hbm, tokf_hbm, len_hbm, tid_hbm, did_hbm, dom_hbm, hrs_hbm,
             kar_hbm, des_hbm, w2vf_hbm, typet_hbm, dayt_hbm, domt_hbm,
             out_hbm, tok_v, tokf_v, emb_v, out_v, lens_v, tids_v, dids_v,
             doms_v, hrs_v, kar_v, des_v, typet_v, dayt_v, domt_v, row0_v,
             row0u_v, sem):
    wid = lax.axis_index("s") * _NC + lax.axis_index("c")
    base = wid * RPW
    w2v_hbm = w2vf_hbm

    pltpu.sync_copy(len_hbm.at[pl.ds(base, RPW)], lens_v)
    pltpu.sync_copy(tid_hbm.at[pl.ds(base, RPW)], tids_v)
    pltpu.sync_copy(did_hbm.at[pl.ds(base, RPW)], dids_v)
    pltpu.sync_copy(dom_hbm.at[pl.ds(base, RPW)], doms_v)
    pltpu.sync_copy(hrs_hbm.at[pl.ds(base, RPW)], hrs_v)
    pltpu.sync_copy(kar_hbm.at[pl.ds(base, RPW)], kar_v)
    pltpu.sync_copy(des_hbm.at[pl.ds(base, RPW)], des_v)
    pltpu.sync_copy(typet_hbm, typet_v)
    pltpu.sync_copy(dayt_hbm, dayt_v)
    pltpu.sync_copy(domt_hbm, domt_v)
    pltpu.sync_copy(w2v_hbm.at[pl.ds(0, 1), :], row0_v)

    iota16 = lax.iota(jnp.int32, 16)
    f32 = jnp.float32

    def zero_body(i, c):
        out_v[pl.ds(i * 16, 16)] = jnp.zeros((16,), f32)
        return c

    lax.fori_loop(0, 256, zero_body, 0)
    rowpart16 = (iota16 // 8) * 2048 + (iota16 % 8) * 128

    for c in range(7):
        w0 = plsc.bitcast(row0_v[0, pl.ds(16 * c, 16)], jnp.int32)
        row0u_v[pl.ds(32 * c, 16)] = plsc.bitcast(w0 << 16, f32)
        row0u_v[pl.ds(32 * c + 16, 16)] = plsc.bitcast(w0 & -65536, f32)

    def group_body(g, carry):
        gbase = base + g * 16
        off = g * 16
        pltpu.sync_copy(tok_hbm.at[pl.ds(gbase, 16), :], tok_v)
        pltpu.sync_copy(tokf_hbm.at[pl.ds(gbase * L, 16 * L)], tokf_v)
        inv16 = 1.0 / jnp.maximum(
            lens_v[pl.ds(off, 16)].astype(jnp.float32), 1.0)

        pltpu.async_copy(w2v_hbm.at[tok_v.at[0]], emb_v.at[0], sem.at[0])
        pltpu.async_copy(w2v_hbm.at[tok_v.at[1]], emb_v.at[1], sem.at[1])

        def row_body(r, carry2):
            par = lax.rem(r, 3)
            nxt = lax.rem(r + 2, 3)

            @pl.when(r < 14)
            def _():
                pltpu.async_copy(w2v_hbm.at[tok_v.at[r + 2]], emb_v.at[nxt],
                                 sem.at[nxt])

            pltpu.make_async_copy(w2v_hbm.at[tok_v.at[r]], emb_v.at[par],
                                  sem.at[par]).wait()

            def acc_body(j, acc):
                new = []
                for c in range(7):
                    w = plsc.bitcast(emb_v[par, j, pl.ds(16 * c, 16)],
                                     jnp.int32)
                    new.append(acc[2 * c] + plsc.bitcast(w << 16, f32))
                    new.append(acc[2 * c + 1]
                               + plsc.bitcast(w & -65536, f32))
                return tuple(new)

            acc = lax.fori_loop(
                0, L, acc_body,
                tuple(jnp.zeros((16,), f32) for _ in range(14)))

            n0 = jnp.zeros((16,), jnp.int32)
            rsplat = jnp.full((16,), r, jnp.int32)
            for c in range(4):
                colpos = c * 16 + iota16
                colidx = jnp.minimum(colpos, L - 1)
                vals = plsc.load_gather(tokf_v, [rsplat * L + colidx])
                n0 = n0 + plsc.all_reduce_population_count(
                    (vals == 0) & (colpos < L))
            n0f = n0.astype(f32)
            inv = inv16.at[rsplat].get(mode="promise_in_bounds")

            rowoff = (r // 8) * 2048 + lax.rem(r, 8) * 128
            sidx = jnp.full((16,), rowoff, jnp.int32) + 2 * iota16
            for c in range(7):
                va = (acc[2 * c] - n0f * row0u_v[pl.ds(32 * c, 16)]) * inv
                vb = (acc[2 * c + 1]
                      - n0f * row0u_v[pl.ds(32 * c + 16, 16)]) * inv
                dbase = (1024 + 32 * c - 128) if c >= 4 else 32 * c
                plsc.store_scatter(out_v, [sidx + dbase], va)
                plsc.store_scatter(out_v, [sidx + (dbase + 1)], vb)
            return carry2

        lax.fori_loop(0, 16, row_body, 0)

        tid = tids_v[pl.ds(off, 16)]
        did = dids_v[pl.ds(off, 16)]
        dmid = doms_v[pl.ds(off, 16)]
        for c in range(8):
            v = plsc.load_gather(typet_v, [tid * 8 + c])
            plsc.store_scatter(out_v, [rowpart16 + (896 + 200 + c)], v)
        for c in range(3):
            v = plsc.load_gather(dayt_v, [did * 3 + c])
            plsc.store_scatter(out_v, [rowpart16 + (896 + 208 + c)], v)
        for c in range(8):
            v = plsc.load_gather(domt_v, [dmid * 8 + c])
            plsc.store_scatter(out_v, [rowpart16 + (896 + 211 + c)], v)
        plsc.store_scatter(out_v, [rowpart16 + (896 + 219)],
                           hrs_v[pl.ds(off, 16)])
        plsc.store_scatter(out_v, [rowpart16 + (896 + 220)],
                           kar_v[pl.ds(off, 16)])
        plsc.store_scatter(out_v, [rowpart16 + (896 + 221)],
                           des_v[pl.ds(off, 16)])

        pltpu.sync_copy(out_v, out_hbm.at[pl.ds(gbase * 256, 4096)])
        return carry

    lax.fori_loop(0, GPW, group_body, 0)


def _sc_pool(tok, lens, tids, dids, doms, hours, karmas, desc,
             w2v, typet, dayt, domt):
    mesh = plsc.VectorSubcoreMesh(core_axis_name="c", subcore_axis_name="s")
    fn = pl.kernel(
        _sc_body,
        mesh=mesh,
        compiler_params=pltpu.CompilerParams(needs_layout_passes=False,
                                             use_tc_tiling_on_sc=False),
        out_type=jax.ShapeDtypeStruct((B * 256,), jnp.float32),
        scratch_types=[
            pltpu.VMEM((16, L), jnp.int32),
            pltpu.VMEM((16 * L,), jnp.int32),
            pltpu.VMEM((3, L, 128), jnp.float32),
            pltpu.VMEM((4096,), jnp.float32),
            pltpu.VMEM((RPW,), jnp.int32),
            pltpu.VMEM((RPW,), jnp.int32),
            pltpu.VMEM((RPW,), jnp.int32),
            pltpu.VMEM((RPW,), jnp.int32),
            pltpu.VMEM((RPW,), jnp.float32),
            pltpu.VMEM((RPW,), jnp.float32),
            pltpu.VMEM((RPW,), jnp.float32),
            pltpu.VMEM((NT * 8,), jnp.float32),
            pltpu.VMEM((ND * 3,), jnp.float32),
            pltpu.VMEM((NDOM * 8,), jnp.float32),
            pltpu.VMEM((1, 128), jnp.float32),
            pltpu.VMEM((DP,), jnp.float32),
            pltpu.SemaphoreType.DMA((3,)),
        ],
    )
    return fn(tok, tok.reshape(-1), lens, tids, dids, doms, hours, karmas,
              desc, w2v, typet, dayt, domt)


DP = 224
DW = 112


def _tr_body(x_ref, o_ref):
    xt = x_ref[...].T
    n = xt.shape[0]
    xt = jnp.concatenate([xt, jnp.zeros((n, DP - D), jnp.float32)], axis=1)
    i0 = lax.broadcasted_iota(jnp.int32, (DP, DW), 0)
    i1 = lax.broadcasted_iota(jnp.int32, (DP, DW), 1)
    ev = jnp.dot(xt, (i0 == 2 * i1).astype(jnp.float32),
                 preferred_element_type=jnp.float32)
    od = jnp.dot(xt, (i0 == 2 * i1 + 1).astype(jnp.float32),
                 preferred_element_type=jnp.float32)
    ev16 = lax.bitcast_convert_type(ev.astype(jnp.bfloat16), jnp.uint16)
    od16 = lax.bitcast_convert_type(od.astype(jnp.bfloat16), jnp.uint16)
    w = ev16.astype(jnp.uint32) | (od16.astype(jnp.uint32) << 16)
    w = jnp.concatenate([w, jnp.zeros((n, 128 - DW), jnp.uint32)], axis=1)
    o_ref[...] = lax.bitcast_convert_type(w, jnp.float32)


def _tr(w2v_t):
    blk = 2048
    return pl.pallas_call(
        _tr_body,
        grid=((V + blk - 1) // blk,),
        in_specs=[pl.BlockSpec((D, blk), lambda i: (0, i))],
        out_specs=pl.BlockSpec((blk, 128), lambda i: (i, 0)),
        out_shape=jax.ShapeDtypeStruct((V, 128), jnp.float32),
    )(w2v_t)


def _mlp_body(x_ref, w1_ref, b1_ref, w2_ref, b2_ref, o_ref):
    n = x_ref.shape[0]
    h2 = jnp.dot(x_ref[...], w1_ref[...],
                 preferred_element_type=jnp.float32)
    h = h2[0:n - 8, 0:64] + h2[8:n, 64:128]
    h = jnp.maximum(h + b1_ref[...][None, :], 0.0)
    o = jnp.sum(h * w2_ref[...][None, :], axis=1) + b2_ref[...]
    o_ref[...] = jnp.concatenate([o, jnp.zeros((8,), jnp.float32)])


def _mlp(fused_tiles, w1cat, b1, w2row, b2):
    blk = 4096
    return pl.pallas_call(
        _mlp_body,
        grid=(B * 2 // blk,),
        in_specs=[
            pl.BlockSpec((blk, 128), lambda i: (i, 0)),
            pl.BlockSpec((128, 128), lambda i: (0, 0)),
            pl.BlockSpec((64,), lambda i: (0,)),
            pl.BlockSpec((64,), lambda i: (0,)),
            pl.BlockSpec((1,), lambda i: (0,)),
        ],
        out_specs=pl.BlockSpec((blk,), lambda i: (i,)),
        out_shape=jax.ShapeDtypeStruct((B * 2,), jnp.float32),
    )(fused_tiles, w1cat, b1, w2row, b2)


def kernel(title_tokens, lengths, type_ids, day_ids, domain_ids, hours,
           karmas, descendants, w2v_weight, type_table, day_table,
           domain_table, W1, b1, W2, b2):
    tok = title_tokens.astype(jnp.int32)
    w2v_weight = _tr(w2v_weight.T)
    fused = _sc_pool(
        tok, lengths.astype(jnp.int32), type_ids.astype(jnp.int32),
        day_ids.astype(jnp.int32), domain_ids.astype(jnp.int32),
        hours, karmas, descendants, w2v_weight,
        type_table.reshape(-1), day_table.reshape(-1),
        domain_table.reshape(-1)).reshape(B * 2, 128)
    w1t = W1.T
    w1cat = jnp.concatenate(
        [w1t[0:128], jnp.pad(w1t[128:], ((0, 34), (0, 0)))], axis=1)
    out_raw = _mlp(fused, w1cat, b1, W2.reshape(-1), b2)
    return out_raw.reshape(B // 8, 16)[:, 0:8].reshape(-1)


# speedup vs baseline: 1.5085x; 1.0603x over previous
import functools

import jax
import jax.numpy as jnp
from jax import lax
from jax.experimental import pallas as pl
from jax.experimental.pallas import tpu as pltpu
from jax.experimental.pallas import tpu_sc as plsc

B = 16384
L = 50
V = 100000
D = 200
NT = 16
ND = 7
NDOM = 10000
FPAD = 224

_INFO = plsc.get_sparse_core_info()
_NC = _INFO.num_cores
_NS = _INFO.num_subcores
NW = _NC * _NS
RPW = B // NW
GPW = RPW // 16


def _sc_body(tok_hbm, tokf_hbm, len_hbm, tid_hbm, did_hbm, dom_hbm, hrs_hbm,
             kar_hbm, des_hbm, w2vf_hbm, typet_hbm, dayt_hbm, domt_hbm,
             out_hbm, tok_v, tokf_v, emb_v, out_v, lens_v, tids_v, dids_v,
             doms_v, hrs_v, kar_v, des_v, typet_v, dayt_v, domt_v, row0_v,
             row0u_v, sem):
    wid = lax.axis_index("s") * _NC + lax.axis_index("c")
    base = wid * RPW
    w2v_hbm = w2vf_hbm

    pltpu.sync_copy(len_hbm.at[pl.ds(base, RPW)], lens_v)
    pltpu.sync_copy(tid_hbm.at[pl.ds(base, RPW)], tids_v)
    pltpu.sync_copy(did_hbm.at[pl.ds(base, RPW)], dids_v)
    pltpu.sync_copy(dom_hbm.at[pl.ds(base, RPW)], doms_v)
    pltpu.sync_copy(hrs_hbm.at[pl.ds(base, RPW)], hrs_v)
    pltpu.sync_copy(kar_hbm.at[pl.ds(base, RPW)], kar_v)
    pltpu.sync_copy(des_hbm.at[pl.ds(base, RPW)], des_v)
    pltpu.sync_copy(typet_hbm, typet_v)
    pltpu.sync_copy(dayt_hbm, dayt_v)
    pltpu.sync_copy(domt_hbm, domt_v)
    pltpu.sync_copy(w2v_hbm.at[pl.ds(0, 1), :], row0_v)

    iota16 = lax.iota(jnp.int32, 16)
    f32 = jnp.float32

    def zero_body(i, c):
        out_v[pl.ds(i * 16, 16)] = jnp.zeros((16,), f32)
        return c

    lax.fori_loop(0, 256, zero_body, 0)
    rowpart16 = (iota16 // 8) * 2048 + (iota16 % 8) * 128

    for c in range(7):
        w0 = plsc.bitcast(row0_v[0, pl.ds(16 * c, 16)], jnp.int32)
        row0u_v[pl.ds(32 * c, 16)] = plsc.bitcast(w0 << 16, f32)
        row0u_v[pl.ds(32 * c + 16, 16)] = plsc.bitcast(w0 & -65536, f32)

    def group_body(g, carry):
        gbase = base + g * 16
        off = g * 16
        pltpu.sync_copy(tok_hbm.at[pl.ds(gbase, 16), :], tok_v)
        pltpu.sync_copy(tokf_hbm.at[pl.ds(gbase * L, 16 * L)], tokf_v)
        inv16 = 1.0 / jnp.maximum(
            lens_v[pl.ds(off, 16)].astype(jnp.float32), 1.0)

        pltpu.async_copy(w2v_hbm.at[tok_v.at[0]], emb_v.at[0], sem.at[0])
        pltpu.async_copy(w2v_hbm.at[tok_v.at[1]], emb_v.at[1], sem.at[1])
        pltpu.async_copy(w2v_hbm.at[tok_v.at[2]], emb_v.at[2], sem.at[2])

        def row_body(r, carry2):
            par = lax.rem(r, 4)
            nxt = lax.rem(r + 3, 4)

            @pl.when(r < 13)
            def _():
                pltpu.async_copy(w2v_hbm.at[tok_v.at[r + 3]], emb_v.at[nxt],
                                 sem.at[nxt])

            pltpu.make_async_copy(w2v_hbm.at[tok_v.at[r]], emb_v.at[par],
                                  sem.at[par]).wait()

            def acc_body(j, acc):
                new = []
                for c in range(7):
                    w = plsc.bitcast(emb_v[par, j, pl.ds(16 * c, 16)],
                                     jnp.int32)
                    new.append(acc[2 * c] + plsc.bitcast(w << 16, f32))
                    new.append(acc[2 * c + 1]
                               + plsc.bitcast(w & -65536, f32))
                return tuple(new)

            acc = lax.fori_loop(
                0, L, acc_body,
                tuple(jnp.zeros((16,), f32) for _ in range(14)))

            n0 = jnp.zeros((16,), jnp.int32)
            rsplat = jnp.full((16,), r, jnp.int32)
            for c in range(4):
                colpos = c * 16 + iota16
                colidx = jnp.minimum(colpos, L - 1)
                vals = plsc.load_gather(tokf_v, [rsplat * L + colidx])
                n0 = n0 + plsc.all_reduce_population_count(
                    (vals == 0) & (colpos < L))
            n0f = n0.astype(f32)
            inv = inv16.at[rsplat].get(mode="promise_in_bounds")

            rowoff = (r // 8) * 2048 + lax.rem(r, 8) * 128
            sidx = jnp.full((16,), rowoff, jnp.int32) + 2 * iota16
            for c in range(7):
                va = (acc[2 * c] - n0f * row0u_v[pl.ds(32 * c, 16)]) * inv
                vb = (acc[2 * c + 1]
                      - n0f * row0u_v[pl.ds(32 * c + 16, 16)]) * inv
                dbase = (1024 + 32 * c - 128) if c >= 4 else 32 * c
                plsc.store_scatter(out_v, [sidx + dbase], va)
                plsc.store_scatter(out_v, [sidx + (dbase + 1)], vb)
            return carry2

        lax.fori_loop(0, 16, row_body, 0)

        tid = tids_v[pl.ds(off, 16)]
        did = dids_v[pl.ds(off, 16)]
        dmid = doms_v[pl.ds(off, 16)]
        for c in range(8):
            v = plsc.load_gather(typet_v, [tid * 8 + c])
            plsc.store_scatter(out_v, [rowpart16 + (896 + 200 + c)], v)
        for c in range(3):
            v = plsc.load_gather(dayt_v, [did * 3 + c])
            plsc.store_scatter(out_v, [rowpart16 + (896 + 208 + c)], v)
        for c in range(8):
            v = plsc.load_gather(domt_v, [dmid * 8 + c])
            plsc.store_scatter(out_v, [rowpart16 + (896 + 211 + c)], v)
        plsc.store_scatter(out_v, [rowpart16 + (896 + 219)],
                           hrs_v[pl.ds(off, 16)])
        plsc.store_scatter(out_v, [rowpart16 + (896 + 220)],
                           kar_v[pl.ds(off, 16)])
        plsc.store_scatter(out_v, [rowpart16 + (896 + 221)],
                           des_v[pl.ds(off, 16)])

        pltpu.sync_copy(out_v, out_hbm.at[pl.ds(gbase * 256, 4096)])
        return carry

    lax.fori_loop(0, GPW, group_body, 0)


def _sc_pool(tok, lens, tids, dids, doms, hours, karmas, desc,
             w2v, typet, dayt, domt):
    mesh = plsc.VectorSubcoreMesh(core_axis_name="c", subcore_axis_name="s")
    fn = pl.kernel(
        _sc_body,
        mesh=mesh,
        compiler_params=pltpu.CompilerParams(needs_layout_passes=False,
                                             use_tc_tiling_on_sc=False),
        out_type=jax.ShapeDtypeStruct((B * 256,), jnp.float32),
        scratch_types=[
            pltpu.VMEM((16, L), jnp.int32),
            pltpu.VMEM((16 * L,), jnp.int32),
            pltpu.VMEM((4, L, 128), jnp.float32),
            pltpu.VMEM((4096,), jnp.float32),
            pltpu.VMEM((RPW,), jnp.int32),
            pltpu.VMEM((RPW,), jnp.int32),
            pltpu.VMEM((RPW,), jnp.int32),
            pltpu.VMEM((RPW,), jnp.int32),
            pltpu.VMEM((RPW,), jnp.float32),
            pltpu.VMEM((RPW,), jnp.float32),
            pltpu.VMEM((RPW,), jnp.float32),
            pltpu.VMEM((NT * 8,), jnp.float32),
            pltpu.VMEM((ND * 3,), jnp.float32),
            pltpu.VMEM((NDOM * 8,), jnp.float32),
            pltpu.VMEM((1, 128), jnp.float32),
            pltpu.VMEM((DP,), jnp.float32),
            pltpu.SemaphoreType.DMA((4,)),
        ],
    )
    return fn(tok, tok.reshape(-1), lens, tids, dids, doms, hours, karmas,
              desc, w2v, typet, dayt, domt)


DP = 224
DW = 112


def _tr_body(x_ref, o_ref):
    xt = x_ref[...].T
    n = xt.shape[0]
    xt = jnp.concatenate([xt, jnp.zeros((n, DP - D), jnp.float32)], axis=1)
    i0 = lax.broadcasted_iota(jnp.int32, (DP, DW), 0)
    i1 = lax.broadcasted_iota(jnp.int32, (DP, DW), 1)
    ev = jnp.dot(xt, (i0 == 2 * i1).astype(jnp.float32),
                 preferred_element_type=jnp.float32)
    od = jnp.dot(xt, (i0 == 2 * i1 + 1).astype(jnp.float32),
                 preferred_element_type=jnp.float32)
    ev16 = lax.bitcast_convert_type(ev.astype(jnp.bfloat16), jnp.uint16)
    od16 = lax.bitcast_convert_type(od.astype(jnp.bfloat16), jnp.uint16)
    w = ev16.astype(jnp.uint32) | (od16.astype(jnp.uint32) << 16)
    w = jnp.concatenate([w, jnp.zeros((n, 128 - DW), jnp.uint32)], axis=1)
    o_ref[...] = lax.bitcast_convert_type(w, jnp.float32)


def _tr(w2v_t):
    blk = 2048
    return pl.pallas_call(
        _tr_body,
        grid=((V + blk - 1) // blk,),
        in_specs=[pl.BlockSpec((D, blk), lambda i: (0, i))],
        out_specs=pl.BlockSpec((blk, 128), lambda i: (i, 0)),
        out_shape=jax.ShapeDtypeStruct((V, 128), jnp.float32),
    )(w2v_t)


def _mlp_body(x_ref, w1_ref, b1_ref, w2_ref, b2_ref, o_ref):
    n = x_ref.shape[0]
    h2 = jnp.dot(x_ref[...], w1_ref[...],
                 preferred_element_type=jnp.float32)
    h = h2[0:n - 8, 0:64] + h2[8:n, 64:128]
    h = jnp.maximum(h + b1_ref[...][None, :], 0.0)
    o = jnp.sum(h * w2_ref[...][None, :], axis=1) + b2_ref[...]
    o_ref[...] = jnp.concatenate([o, jnp.zeros((8,), jnp.float32)])


def _mlp(fused_tiles, w1cat, b1, w2row, b2):
    blk = 4096
    return pl.pallas_call(
        _mlp_body,
        grid=(B * 2 // blk,),
        in_specs=[
            pl.BlockSpec((blk, 128), lambda i: (i, 0)),
            pl.BlockSpec((128, 128), lambda i: (0, 0)),
            pl.BlockSpec((64,), lambda i: (0,)),
            pl.BlockSpec((64,), lambda i: (0,)),
            pl.BlockSpec((1,), lambda i: (0,)),
        ],
        out_specs=pl.BlockSpec((blk,), lambda i: (i,)),
        out_shape=jax.ShapeDtypeStruct((B * 2,), jnp.float32),
    )(fused_tiles, w1cat, b1, w2row, b2)


def kernel(title_tokens, lengths, type_ids, day_ids, domain_ids, hours,
           karmas, descendants, w2v_weight, type_table, day_table,
           domain_table, W1, b1, W2, b2):
    tok = title_tokens.astype(jnp.int32)
    w2v_weight = _tr(w2v_weight.T)
    fused = _sc_pool(
        tok, lengths.astype(jnp.int32), type_ids.astype(jnp.int32),
        day_ids.astype(jnp.int32), domain_ids.astype(jnp.int32),
        hours, karmas, descendants, w2v_weight,
        type_table.reshape(-1), day_table.reshape(-1),
        domain_table.reshape(-1)).reshape(B * 2, 128)
    w1t = W1.T
    w1cat = jnp.concatenate(
        [w1t[0:128], jnp.pad(w1t[128:], ((0, 34), (0, 0)))], axis=1)
    out_raw = _mlp(fused, w1cat, b1, W2.reshape(-1), b2)
    return out_raw.reshape(B // 8, 16)[:, 0:8].reshape(-1)
